# Initial kernel scaffold; baseline (speedup 1.0000x reference)
import jax
import jax.numpy as jnp
from jax.experimental import pallas as pl


def kernel(x, edge_index, Wn1, Wr1, br1, Wn2, Wr2, br2):
    raise NotImplementedError("write your pallas kernel here")


# speedup vs baseline: 3.0210x; 3.0210x over previous
import functools

import jax
import jax.numpy as jnp
from jax import lax
from jax.experimental import pallas as pl
from jax.experimental.pallas import tpu as pltpu
from jax.experimental.pallas import tpu_sc as plsc

_NS = 16
_CB = 80
_BN = 2000
_NL = 16
_NP = 3


def _dot_t(a, w):
    return lax.dot_general(a, w, (((1,), (1,)), ((), ())),
                           preferred_element_type=jnp.float32)


def _matmul2_call(x, wn, wr, br):
    n, d = x.shape

    def body(x_ref, wn_ref, wr_ref, b_ref, y_ref, r_ref):
        xb = x_ref[...]
        y_ref[...] = _dot_t(xb, wn_ref[...])
        r_ref[...] = _dot_t(xb, wr_ref[...]) + b_ref[...]

    return pl.pallas_call(
        body,
        grid=(n // _BN,),
        in_specs=[
            pl.BlockSpec((_BN, d), lambda i: (i, 0)),
            pl.BlockSpec((d, d), lambda i: (0, 0)),
            pl.BlockSpec((d, d), lambda i: (0, 0)),
            pl.BlockSpec((1, d), lambda i: (0, 0)),
        ],
        out_specs=[pl.BlockSpec((_BN, d), lambda i: (i, 0))] * 2,
        out_shape=[jax.ShapeDtypeStruct((n, d), jnp.float32)] * 2,
    )(x, wn, wr, br.reshape(1, d))


def _combine_block(r, a, b, cnt):
    agg = jnp.concatenate([a, b], axis=1)
    dnm = jnp.maximum(cnt[:, 0:1], 1.0)
    o = r + agg / dnm
    nrm = jnp.maximum(jnp.sqrt(jnp.sum(o * o, axis=1, keepdims=True)), 1e-12)
    return o / nrm


def _combine_mm_call(r, a, b, cnt, wn, wr, br):
    n, d = r.shape
    dh = a.shape[1]
    nc = cnt.shape[1]

    def body(r_ref, a_ref, b_ref, c_ref, wn_ref, wr_ref, bias_ref,
             y_ref, r2_ref):
        h = jnp.maximum(
            _combine_block(r_ref[...], a_ref[...], b_ref[...], c_ref[...]),
            0.0)
        y_ref[...] = _dot_t(h, wn_ref[...])
        r2_ref[...] = _dot_t(h, wr_ref[...]) + bias_ref[...]

    return pl.pallas_call(
        body,
        grid=(n // _BN,),
        in_specs=[
            pl.BlockSpec((_BN, d), lambda i: (i, 0)),
            pl.BlockSpec((_BN, dh), lambda i: (i, 0)),
            pl.BlockSpec((_BN, dh), lambda i: (i, 0)),
            pl.BlockSpec((_BN, nc), lambda i: (i, 0)),
            pl.BlockSpec((d, d), lambda i: (0, 0)),
            pl.BlockSpec((d, d), lambda i: (0, 0)),
            pl.BlockSpec((1, d), lambda i: (0, 0)),
        ],
        out_specs=[pl.BlockSpec((_BN, d), lambda i: (i, 0))] * 2,
        out_shape=[jax.ShapeDtypeStruct((n, d), jnp.float32)] * 2,
    )(r, a, b, cnt, wn, wr, br.reshape(1, d))


def _combine_call(r, a, b, cnt):
    n, d = r.shape
    dh = a.shape[1]
    nc = cnt.shape[1]

    def body(r_ref, a_ref, b_ref, c_ref, out_ref):
        out_ref[...] = _combine_block(r_ref[...], a_ref[...], b_ref[...],
                                      c_ref[...])

    return pl.pallas_call(
        body,
        grid=(n // _BN,),
        in_specs=[
            pl.BlockSpec((_BN, d), lambda i: (i, 0)),
            pl.BlockSpec((_BN, dh), lambda i: (i, 0)),
            pl.BlockSpec((_BN, dh), lambda i: (i, 0)),
            pl.BlockSpec((_BN, nc), lambda i: (i, 0)),
        ],
        out_specs=pl.BlockSpec((_BN, d), lambda i: (i, 0)),
        out_shape=jax.ShapeDtypeStruct((n, d), jnp.float32),
    )(r, a, b, cnt)


_SEG_CACHE = {}


def _segsum_call(y2, src_a, src_b, dst3, zrows, with_counts):
    key = (y2.shape, dst3.shape, with_counts)
    if key not in _SEG_CACHE:
        _SEG_CACHE[key] = _build_segsum(y2.shape, dst3.shape, with_counts)
    return _SEG_CACHE[key](y2, src_a, src_b, dst3, zrows)


def _build_segsum(y2_shape, dst3_shape, with_counts):
    n = y2_shape[0] // 2
    dh = y2_shape[1]
    ns, nch, cb = dst3_shape
    nv = -(n // (-8 * _NP)) * 8
    napad = nv + 8
    rpz = napad // ns // 8 * 8
    lastz = napad - (ns - 1) * rpz
    lastv = nv - (ns - 1) * rpz
    nvec = cb // _NL
    assert nch % 2 == 1, "pipelined loop below assumes an odd chunk count"
    assert (_NP - 1) * nv < n <= _NP * nv and n % 8 == 0 and rpz > 0
    assert 0 < lastv <= lastz
    for p in range(_NP):
        vp = min(nv, n - p * nv)
        assert vp - (ns - 1) * rpz > 0 and vp % 8 == 0

    mesh = plsc.VectorSubcoreMesh(core_axis_name="c", subcore_axis_name="s")

    out_type = [jax.ShapeDtypeStruct((n, dh), jnp.float32),
                jax.ShapeDtypeStruct((n, dh), jnp.float32)]
    if with_counts:
        out_type.append(jax.ShapeDtypeStruct((_NP * nv, dh), jnp.float32))

    @functools.partial(
        pl.kernel,
        mesh=mesh,
        out_type=out_type,
        scratch_types=[
            pltpu.VMEM((nch, cb), jnp.int32),
            pltpu.VMEM((nch, cb), jnp.int32),
            pltpu.VMEM((cb, dh), jnp.float32),
            pltpu.VMEM((cb, dh), jnp.float32),
            pltpu.VMEM_SHARED((napad, dh), jnp.float32),
            pltpu.SemaphoreType.DMA,
            pltpu.SemaphoreType.DMA,
        ],
    )
    def seg(y_ref, sa_ref, sb_ref, d_ref, z_ref, agg_a, agg_b, *rest):
        if with_counts:
            cnt_out = rest[0]
            rest = rest[1:]
        (sidx, didx_t, rows_a, rows_b, acc, sem_a, sem_b) = rest
        c = lax.axis_index("c")
        s = lax.axis_index("s")
        arow0 = s * rpz

        @pl.when(c == 0)
        def _():
            pltpu.sync_copy(sa_ref.at[s], sidx)

        @pl.when(c == 1)
        def _():
            pltpu.sync_copy(sb_ref.at[s], sidx)

        def load_transform(lo):
            pltpu.sync_copy(d_ref.at[s], didx_t)

            def transform(j, carry):
                for k in range(nvec):
                    sl = pl.ds(k * _NL, _NL)
                    v = didx_t[j, sl]
                    g = nv + (v & 7)
                    inr = (v >= lo) & (v < lo + nv)
                    didx_t[j, sl] = jnp.where(inr, v - lo, g)
                return carry

            lax.fori_loop(0, nch, transform, 0)

        def zero_acc():
            @pl.when(s < ns - 1)
            def _():
                pltpu.sync_copy(z_ref.at[pl.ds(0, rpz)],
                                acc.at[pl.ds(arow0, rpz)])

            @pl.when(s == ns - 1)
            def _():
                pltpu.sync_copy(z_ref.at[pl.ds(0, lastz)],
                                acc.at[pl.ds(arow0, lastz)])

        def fire(j, buf, sem):
            pltpu.async_copy(y_ref.at[sidx.at[j]], buf, sem)

        def wait(buf, sem):
            pltpu.make_async_copy(y_ref.at[pl.ds(0, cb)], buf, sem).wait()

        def scat(j, buf):
            pltpu.sync_copy(buf, acc.at[didx_t.at[j]], add=True)

        for p in range(_NP):
            vp = min(nv, n - p * nv)
            lastw = vp - (ns - 1) * rpz
            load_transform(p * nv)
            zero_acc()
            plsc.subcore_barrier()

            fire(0, rows_a, sem_a)

            def step(jj, carry):
                j0 = 2 * jj
                fire(j0 + 1, rows_b, sem_b)
                wait(rows_a, sem_a)
                scat(j0, rows_a)
                fire(j0 + 2, rows_a, sem_a)
                wait(rows_b, sem_b)
                scat(j0 + 1, rows_b)
                return carry

            lax.fori_loop(0, (nch - 1) // 2, step, 0)
            wait(rows_a, sem_a)
            scat(nch - 1, rows_a)

            plsc.subcore_barrier()
            orow0 = p * nv + arow0

            def wout(dst_ref):
                @pl.when(s < ns - 1)
                def _():
                    pltpu.sync_copy(acc.at[pl.ds(arow0, rpz)],
                                    dst_ref.at[pl.ds(orow0, rpz)])

                @pl.when(s == ns - 1)
                def _():
                    pltpu.sync_copy(acc.at[pl.ds(arow0, lastw)],
                                    dst_ref.at[pl.ds(orow0, lastw)])

            @pl.when(c == 0)
            def _():
                wout(agg_a)

            @pl.when(c == 1)
            def _():
                wout(agg_b)

        if with_counts:
            def fill_ones(r, carry):
                for k in range(dh // _NL):
                    rows_a[r, pl.ds(k * _NL, _NL)] = jnp.ones((_NL,),
                                                              jnp.float32)
                return carry

            lax.fori_loop(0, cb, fill_ones, 0)

            for q01 in range(2):
                q = jnp.minimum(c * 2 + q01, _NP - 1)
                load_transform(q * nv)
                zero_acc()
                plsc.subcore_barrier()

                def cstep(j, carry):
                    scat(j, rows_a)
                    return carry

                lax.fori_loop(0, nch, cstep, 0)
                plsc.subcore_barrier()
                crow0 = q * nv + arow0

                @pl.when(s < ns - 1)
                def _():
                    pltpu.sync_copy(acc.at[pl.ds(arow0, rpz)],
                                    cnt_out.at[pl.ds(crow0, rpz)])

                @pl.when(s == ns - 1)
                def _():
                    pltpu.sync_copy(acc.at[pl.ds(arow0, lastv)],
                                    cnt_out.at[pl.ds(crow0, lastv)])

    return seg


def kernel(x, edge_index, Wn1, Wr1, br1, Wn2, Wr2, br2):
    n, d = x.shape
    e = edge_index.shape[1]
    dh = d // 2
    ns, cb = _NS, _CB
    ep = e // ns
    nch = ep // cb
    nv = -(n // (-8 * _NP)) * 8
    napad = nv + 8
    rpz = napad // ns // 8 * 8
    lastz = napad - (ns - 1) * rpz

    ei = edge_index.astype(jnp.int32)
    src2 = ei[0] * 2
    src_a = src2.reshape(ns, nch, cb)
    src_b = (src2 + 1).reshape(ns, nch, cb)
    dst3 = ei[1].reshape(ns, nch, cb)
    zrows = jnp.zeros((lastz, dh), jnp.float32)

    y1, r1 = _matmul2_call(x, Wn1, Wr1, br1)
    a1, b1, cnt = _segsum_call(y1.reshape(2 * n, dh), src_a, src_b, dst3,
                               zrows, with_counts=True)
    y2, r2 = _combine_mm_call(r1, a1, b1, cnt, Wn2, Wr2, br2)
    a2, b2 = _segsum_call(y2.reshape(2 * n, dh), src_a, src_b, dst3,
                          zrows, with_counts=False)
    return _combine_call(r2, a2, b2, cnt)
